# baseline (speedup 1.0000x reference)
import functools

import jax
import jax.numpy as jnp
from jax import lax
from jax.experimental import pallas as pl
from jax.experimental.pallas import tpu as pltpu
from jax.experimental.pallas import tpu_sc as plsc

TOTAL_TOKENS = 32768
D_FEAT = 1024
BATCH = 16
P_DROP = 0.5
LANES = 16

TC_ROWS = 20480
SC_ROWS = TOTAL_TOKENS - TC_ROWS
TC_BLOCK = 512

NUM_WORKERS = 32
ROWS_PER_W = SC_ROWS // NUM_WORKERS
CHUNK = 16
NB = 2
NCHUNKS = ROWS_PER_W // CHUNK
NVEC = D_FEAT // LANES


def _sc_body(x_hbm, idx_hbm, noise_hbm, out_hbm, noise_v, idx_v,
             in0, in1, out0, out1, si0, si1, so0, so1):
    wid = lax.axis_index("s") * 2 + lax.axis_index("c")
    base = TC_ROWS + wid * ROWS_PER_W
    obase = wid * ROWS_PER_W

    inbufs = (in0, in1)
    outbufs = (out0, out1)
    sin = (si0, si1)
    sout = (so0, so1)

    def in_copy(g, b):
        return pltpu.make_async_copy(
            x_hbm.at[pl.ds(base + g * CHUNK, CHUNK)], inbufs[b], sin[b])

    def out_copy(g, b):
        return pltpu.make_async_copy(
            outbufs[b], out_hbm.at[pl.ds(obase + g * CHUNK, CHUNK)], sout[b])

    for b in range(NB):
        in_copy(b, b).start()

    pltpu.sync_copy(noise_hbm, noise_v)
    pltpu.sync_copy(idx_hbm.at[pl.ds(base, ROWS_PER_W)],
                    idx_v.at[pl.ds(0, ROWS_PER_W)])
    idx_v[pl.ds(ROWS_PER_W, LANES)] = idx_v[pl.ds(ROWS_PER_W - LANES, LANES)]

    def compute(b, r0):
        ib, ob = inbufs[b], outbufs[b]
        idx_head = idx_v[pl.ds(r0, LANES)]
        b_first = idx_head[0]
        b_last = idx_head[CHUNK - 1]

        @pl.when(b_first == b_last)
        def _fast():
            def col_body(i, _c):
                sl = pl.ds(i * LANES, LANES)
                nv = noise_v[b_first, sl]
                for r in range(CHUNK):
                    ob[r, sl] = ib[r, sl] * nv
                return _c
            lax.fori_loop(0, NVEC, col_body, None)

        @pl.when(b_first != b_last)
        def _slow():
            for j in range(CHUNK):
                bj = idx_head[j]

                def col_body(i, _c, j=j, bj=bj):
                    sl = pl.ds(i * LANES, LANES)
                    ob[j, sl] = ib[j, sl] * noise_v[bj, sl]
                    return _c
                lax.fori_loop(0, NVEC, col_body, None)

    def group(G, _):
        for b in range(NB):
            g = G * NB + b
            in_copy(g, b).wait()

            @pl.when(G > 0)
            def _drain(g=g, b=b):
                out_copy(g - NB, b).wait()

            compute(b, g * CHUNK)
            out_copy(g, b).start()

            @pl.when(g + NB < NCHUNKS)
            def _prefetch(g=g, b=b):
                in_copy(g + NB, b).start()
        return _

    lax.fori_loop(0, NCHUNKS // NB, group, None)

    for b in range(NB):
        out_copy(NCHUNKS - NB + b, b).wait()


def _sc_part(x, batch_idxs, noise_table):
    mesh = plsc.VectorSubcoreMesh(core_axis_name="c", subcore_axis_name="s")
    f = functools.partial(
        pl.kernel,
        mesh=mesh,
        out_type=jax.ShapeDtypeStruct((SC_ROWS, D_FEAT), jnp.float32),
        scratch_types=[
            pltpu.VMEM((BATCH, D_FEAT), jnp.float32),
            pltpu.VMEM((ROWS_PER_W + LANES,), jnp.int32),
            pltpu.VMEM((CHUNK, D_FEAT), jnp.float32),
            pltpu.VMEM((CHUNK, D_FEAT), jnp.float32),
            pltpu.VMEM((CHUNK, D_FEAT), jnp.float32),
            pltpu.VMEM((CHUNK, D_FEAT), jnp.float32),
            pltpu.SemaphoreType.DMA,
            pltpu.SemaphoreType.DMA,
            pltpu.SemaphoreType.DMA,
            pltpu.SemaphoreType.DMA,
        ],
    )(_sc_body)
    return f(x, batch_idxs, noise_table)


def _tc_body(idx_ref, x_ref, noise_ref, o_ref):
    idx = idx_ref[0, 0]
    onehot = (idx[:, None] == lax.broadcasted_iota(
        jnp.int32, (TC_BLOCK, BATCH), 1)).astype(jnp.float32)
    noise = jnp.dot(onehot, noise_ref[...],
                    preferred_element_type=jnp.float32)
    o_ref[...] = x_ref[...] * noise


def _tc_part(x, batch_idxs, noise_table):
    idx3d = batch_idxs.reshape(TOTAL_TOKENS // TC_BLOCK, 1, TC_BLOCK)
    grid = (TC_ROWS // TC_BLOCK,)
    return pl.pallas_call(
        _tc_body,
        grid=grid,
        in_specs=[
            pl.BlockSpec((1, 1, TC_BLOCK), lambda i: (i, 0, 0)),
            pl.BlockSpec((TC_BLOCK, D_FEAT), lambda i: (i, 0)),
            pl.BlockSpec((BATCH, D_FEAT), lambda i: (0, 0)),
        ],
        out_specs=pl.BlockSpec((TC_BLOCK, D_FEAT), lambda i: (i, 0)),
        out_shape=jax.ShapeDtypeStruct((TC_ROWS, D_FEAT), jnp.float32),
    )(idx3d, x, noise_table)


def kernel(input, batch_idxs):
    keep = jax.random.bernoulli(jax.random.key(42), 1.0 - P_DROP,
                                (BATCH, input.shape[1]))
    noise_table = keep.astype(input.dtype) / (1.0 - P_DROP)

    sc_out = _sc_part(input, batch_idxs, noise_table)
    tc_out = _tc_part(input, batch_idxs, noise_table)
    return jnp.concatenate([tc_out, sc_out], axis=0)


# speedup vs baseline: 1.9594x; 1.9594x over previous
import functools

import jax
import jax.numpy as jnp
from jax import lax
from jax.experimental import pallas as pl
from jax.experimental.pallas import tpu as pltpu
from jax.experimental.pallas import tpu_sc as plsc

TOTAL_TOKENS = 32768
D_FEAT = 1024
BATCH = 16
P_DROP = 0.5
LANES = 16

TC_ROWS = 32768
SC_ROWS = TOTAL_TOKENS - TC_ROWS
TC_BLOCK = 512

NUM_WORKERS = 32
ROWS_PER_W = SC_ROWS // NUM_WORKERS
CHUNK = 16
NB = 2
NCHUNKS = ROWS_PER_W // CHUNK
NVEC = D_FEAT // LANES


def _sc_body(x_hbm, idx_hbm, noise_hbm, out_hbm, noise_v, idx_v,
             in0, in1, out0, out1, si0, si1, so0, so1):
    wid = lax.axis_index("s") * 2 + lax.axis_index("c")
    base = TC_ROWS + wid * ROWS_PER_W
    obase = wid * ROWS_PER_W

    inbufs = (in0, in1)
    outbufs = (out0, out1)
    sin = (si0, si1)
    sout = (so0, so1)

    def in_copy(g, b):
        return pltpu.make_async_copy(
            x_hbm.at[pl.ds(base + g * CHUNK, CHUNK)], inbufs[b], sin[b])

    def out_copy(g, b):
        return pltpu.make_async_copy(
            outbufs[b], out_hbm.at[pl.ds(obase + g * CHUNK, CHUNK)], sout[b])

    for b in range(NB):
        in_copy(b, b).start()

    pltpu.sync_copy(noise_hbm, noise_v)
    pltpu.sync_copy(idx_hbm.at[pl.ds(base, ROWS_PER_W)],
                    idx_v.at[pl.ds(0, ROWS_PER_W)])
    idx_v[pl.ds(ROWS_PER_W, LANES)] = idx_v[pl.ds(ROWS_PER_W - LANES, LANES)]

    def compute(b, r0):
        ib, ob = inbufs[b], outbufs[b]
        idx_head = idx_v[pl.ds(r0, LANES)]
        b_first = idx_head[0]
        b_last = idx_head[CHUNK - 1]

        @pl.when(b_first == b_last)
        def _fast():
            def col_body(i, _c):
                sl = pl.ds(i * LANES, LANES)
                nv = noise_v[b_first, sl]
                for r in range(CHUNK):
                    ob[r, sl] = ib[r, sl] * nv
                return _c
            lax.fori_loop(0, NVEC, col_body, None)

        @pl.when(b_first != b_last)
        def _slow():
            for j in range(CHUNK):
                bj = idx_head[j]

                def col_body(i, _c, j=j, bj=bj):
                    sl = pl.ds(i * LANES, LANES)
                    ob[j, sl] = ib[j, sl] * noise_v[bj, sl]
                    return _c
                lax.fori_loop(0, NVEC, col_body, None)

    def group(G, _):
        for b in range(NB):
            g = G * NB + b
            in_copy(g, b).wait()

            @pl.when(G > 0)
            def _drain(g=g, b=b):
                out_copy(g - NB, b).wait()

            compute(b, g * CHUNK)
            out_copy(g, b).start()

            @pl.when(g + NB < NCHUNKS)
            def _prefetch(g=g, b=b):
                in_copy(g + NB, b).start()
        return _

    lax.fori_loop(0, NCHUNKS // NB, group, None)

    for b in range(NB):
        out_copy(NCHUNKS - NB + b, b).wait()


def _sc_part(x, batch_idxs, noise_table):
    mesh = plsc.VectorSubcoreMesh(core_axis_name="c", subcore_axis_name="s")
    f = functools.partial(
        pl.kernel,
        mesh=mesh,
        out_type=jax.ShapeDtypeStruct((SC_ROWS, D_FEAT), jnp.float32),
        scratch_types=[
            pltpu.VMEM((BATCH, D_FEAT), jnp.float32),
            pltpu.VMEM((ROWS_PER_W + LANES,), jnp.int32),
            pltpu.VMEM((CHUNK, D_FEAT), jnp.float32),
            pltpu.VMEM((CHUNK, D_FEAT), jnp.float32),
            pltpu.VMEM((CHUNK, D_FEAT), jnp.float32),
            pltpu.VMEM((CHUNK, D_FEAT), jnp.float32),
            pltpu.SemaphoreType.DMA,
            pltpu.SemaphoreType.DMA,
            pltpu.SemaphoreType.DMA,
            pltpu.SemaphoreType.DMA,
        ],
    )(_sc_body)
    return f(x, batch_idxs, noise_table)


def _tc_body(idx_ref, x_ref, noise_ref, o_ref):
    idx = idx_ref[0, 0]
    onehot = (idx[:, None] == lax.broadcasted_iota(
        jnp.int32, (TC_BLOCK, BATCH), 1)).astype(jnp.float32)
    noise = jnp.dot(onehot, noise_ref[...],
                    preferred_element_type=jnp.float32)
    o_ref[...] = x_ref[...] * noise


def _tc_part(x, batch_idxs, noise_table):
    idx3d = batch_idxs.reshape(TOTAL_TOKENS // TC_BLOCK, 1, TC_BLOCK)
    grid = (TC_ROWS // TC_BLOCK,)
    return pl.pallas_call(
        _tc_body,
        grid=grid,
        in_specs=[
            pl.BlockSpec((1, 1, TC_BLOCK), lambda i: (i, 0, 0)),
            pl.BlockSpec((TC_BLOCK, D_FEAT), lambda i: (i, 0)),
            pl.BlockSpec((BATCH, D_FEAT), lambda i: (0, 0)),
        ],
        out_specs=pl.BlockSpec((TC_BLOCK, D_FEAT), lambda i: (i, 0)),
        out_shape=jax.ShapeDtypeStruct((TC_ROWS, D_FEAT), jnp.float32),
    )(idx3d, x, noise_table)


def kernel(input, batch_idxs):
    keep = jax.random.bernoulli(jax.random.key(42), 1.0 - P_DROP,
                                (BATCH, input.shape[1]))
    noise_table = keep.astype(input.dtype) / (1.0 - P_DROP)

    return _tc_part(input, batch_idxs, noise_table)
